# baseline (speedup 1.0000x reference)
import functools

import jax
import jax.numpy as jnp
from jax import lax
from jax.experimental import pallas as pl
from jax.experimental.pallas import tpu as pltpu
from jax.experimental.pallas import tpu_sc as plsc

N = 10000
E = 320000
F = 128
NC = 2
NS = 16
NW = NC * NS
K = 128
EDGES_PER_WORKER = E // NW
CHUNKS = 80
EPW_PAD = CHUNKS * K
NPAD = 10240
ROWS_PER_TILE = NPAD // NS
NSLOT = 4

_mesh = plsc.VectorSubcoreMesh(core_axis_name="c", subcore_axis_name="s")


@functools.partial(
    pl.kernel,
    mesh=_mesh,
    out_type=jax.ShapeDtypeStruct((NC * NPAD, F), jnp.float32),
    scratch_types=[
        pltpu.VMEM((NSLOT, K), jnp.int32),
        pltpu.VMEM((NSLOT, K), jnp.int32),
        pltpu.VMEM((K, F), jnp.float32),
        pltpu.VMEM((K, F), jnp.float32),
        pltpu.VMEM_SHARED((NPAD, F), jnp.float32),
        pltpu.SemaphoreType.DMA,
        pltpu.SemaphoreType.DMA,
        pltpu.SemaphoreType.DMA,
        pltpu.SemaphoreType.DMA,
        pltpu.SemaphoreType.DMA,
        pltpu.SemaphoreType.DMA,
    ],
)
def _sc_scatter(feat_hbm, src_hbm, dst_hbm, zeros_hbm, out_hbm,
                src_ring, dst_ring, rows_a, rows_b, acc_sh,
                sem_s0, sem_s1, sem_s2, sem_s3, sem_a, sem_b):
    c = lax.axis_index("c")
    s = lax.axis_index("s")
    w = c * NS + s
    sem_s = [sem_s0, sem_s1, sem_s2, sem_s3]
    rows = [rows_a, rows_b]
    sem_r = [sem_a, sem_b]
    base = w * jnp.int32(EPW_PAD)

    def fire_idx(edge0, slot):
        off = pl.multiple_of(edge0, 8)
        pltpu.async_copy(src_hbm.at[pl.ds(off, K)], src_ring.at[jnp.int32(slot)],
                         sem_s[slot])
        pltpu.async_copy(dst_hbm.at[pl.ds(off, K)], dst_ring.at[jnp.int32(slot)],
                         sem_s[slot])

    def wait_idx(slot):
        pltpu.make_async_copy(src_hbm.at[pl.ds(0, K)],
                              src_ring.at[jnp.int32(slot)], sem_s[slot]).wait()
        pltpu.make_async_copy(dst_hbm.at[pl.ds(0, K)],
                              dst_ring.at[jnp.int32(slot)], sem_s[slot]).wait()

    def fire_gather(buf):
        pltpu.async_copy(feat_hbm.at[src_ring.at[jnp.int32(buf % NSLOT)]],
                         rows[buf % 2], sem_r[buf % 2])

    def wait_gather(buf):
        pltpu.make_async_copy(feat_hbm.at[pl.ds(0, K)], rows[buf % 2],
                              sem_r[buf % 2]).wait()

    for t in range(NSLOT):
        fire_idx(base + jnp.int32(t * K), t)
    pltpu.sync_copy(zeros_hbm,
                    acc_sh.at[pl.ds(s * ROWS_PER_TILE, ROWS_PER_TILE)])
    plsc.subcore_barrier()
    wait_idx(0)
    fire_gather(0)
    wait_idx(1)
    fire_gather(1)

    def body(_, j):
        for t in range(NSLOT):
            jt = j + jnp.int32(t)
            wait_gather(t)
            pltpu.sync_copy(rows[t % 2], acc_sh.at[dst_ring.at[jnp.int32(t)]],
                            add=True)

            @pl.when(jt < jnp.int32(CHUNKS - NSLOT))
            def _():
                fire_idx(base + (jt + jnp.int32(NSLOT)) * jnp.int32(K), t)

            @pl.when(jt < jnp.int32(CHUNKS - 2))
            def _():
                wait_idx((t + 2) % NSLOT)
                fire_gather(t + 2)

        return j + jnp.int32(NSLOT)

    lax.fori_loop(0, CHUNKS // NSLOT, body, jnp.int32(0))
    plsc.subcore_barrier()

    row0 = s * ROWS_PER_TILE
    pltpu.sync_copy(acc_sh.at[pl.ds(row0, ROWS_PER_TILE)],
                    out_hbm.at[pl.ds(c * NPAD + row0, ROWS_PER_TILE)])


def _mm_body(p_ref, w_ref, o_ref):
    o_ref[...] = jnp.dot(p_ref[0] + p_ref[1], w_ref[...],
                         preferred_element_type=jnp.float32)


def _combine_matmul(partial, W):
    return pl.pallas_call(
        _mm_body,
        out_shape=jax.ShapeDtypeStruct((N, F), jnp.float32),
    )(partial, W)


def kernel(features, edge_index, W, a):
    del a
    src = jnp.pad(edge_index[0].astype(jnp.int32).reshape(NW, EDGES_PER_WORKER),
                  ((0, 0), (0, EPW_PAD - EDGES_PER_WORKER)),
                  constant_values=0).reshape(-1)
    dst = jnp.pad(edge_index[1].astype(jnp.int32).reshape(NW, EDGES_PER_WORKER),
                  ((0, 0), (0, EPW_PAD - EDGES_PER_WORKER)),
                  constant_values=N).reshape(-1)
    zeros = jnp.zeros((ROWS_PER_TILE, F), jnp.float32)
    partial = _sc_scatter(features, src, dst, zeros)
    partial = partial.reshape(NC, NPAD, F)[:, :N, :]
    return _combine_matmul(partial, W)


# speedup vs baseline: 1.2459x; 1.2459x over previous
import functools

import jax
import jax.numpy as jnp
from jax import lax
from jax.experimental import pallas as pl
from jax.experimental.pallas import tpu as pltpu
from jax.experimental.pallas import tpu_sc as plsc

N = 10000
E = 320000
F = 128
NC = 2
NS = 16
NW = NC * NS
K = 80
EDGES_PER_WORKER = E // NW
CHUNKS = EDGES_PER_WORKER // K
NPAD = 10240
ROWS_PER_TILE = NPAD // NS

_mesh = plsc.VectorSubcoreMesh(core_axis_name="c", subcore_axis_name="s")


@functools.partial(
    pl.kernel,
    mesh=_mesh,
    out_type=jax.ShapeDtypeStruct((NC * NPAD, F), jnp.float32),
    scratch_types=[
        pltpu.VMEM((K,), jnp.int32),
        pltpu.VMEM((K,), jnp.int32),
        pltpu.VMEM((K, F), jnp.float32),
        pltpu.VMEM_SHARED((NPAD, F), jnp.float32),
        pltpu.SemaphoreType.DMA,
    ],
)
def _sc_scatter(feat_hbm, src_hbm, dst_hbm, zeros_hbm, out_hbm,
                src_v, dst_v, rows_v, acc_sh, sem):
    c = lax.axis_index("c")
    s = lax.axis_index("s")
    w = c * NS + s

    pltpu.sync_copy(zeros_hbm, acc_sh.at[pl.ds(s * ROWS_PER_TILE, ROWS_PER_TILE)])
    plsc.subcore_barrier()

    base = w * jnp.int32(EDGES_PER_WORKER)

    def body(_, off):
        off = pl.multiple_of(off, 8)
        pltpu.sync_copy(src_hbm.at[pl.ds(off, K)], src_v)
        pltpu.sync_copy(dst_hbm.at[pl.ds(off, K)], dst_v)
        pltpu.async_copy(feat_hbm.at[src_v], rows_v, sem).wait()
        pltpu.sync_copy(rows_v, acc_sh.at[dst_v], add=True)
        return off + jnp.int32(K)

    lax.fori_loop(0, CHUNKS, body, base)
    plsc.subcore_barrier()

    row0 = s * ROWS_PER_TILE
    pltpu.sync_copy(acc_sh.at[pl.ds(row0, ROWS_PER_TILE)],
                    out_hbm.at[pl.ds(c * NPAD + row0, ROWS_PER_TILE)])


def _mm_body(p_ref, w_ref, o_ref):
    o_ref[...] = jnp.dot(p_ref[0] + p_ref[1], w_ref[...],
                         preferred_element_type=jnp.float32)


def _combine_matmul(partial, W):
    return pl.pallas_call(
        _mm_body,
        out_shape=jax.ShapeDtypeStruct((N, F), jnp.float32),
    )(partial, W)


def kernel(features, edge_index, W, a):
    del a
    src = edge_index[0].astype(jnp.int32)
    dst = edge_index[1].astype(jnp.int32)
    zeros = jnp.zeros((ROWS_PER_TILE, F), jnp.float32)
    partial = _sc_scatter(features, src, dst, zeros)
    partial = partial.reshape(NC, NPAD, F)[:, :N, :]
    return _combine_matmul(partial, W)


# speedup vs baseline: 1.7639x; 1.4158x over previous
import functools

import jax
import jax.numpy as jnp
from jax import lax
from jax.experimental import pallas as pl
from jax.experimental.pallas import tpu as pltpu
from jax.experimental.pallas import tpu_sc as plsc

N = 10000
E = 320000
F = 128
NC = 2
NS = 16
NW = NC * NS
K = 80
EDGES_PER_WORKER = E // NW
CHUNKS = EDGES_PER_WORKER // K
NPAD = 10240
ROWS_PER_TILE = NPAD // NS

_mesh = plsc.VectorSubcoreMesh(core_axis_name="c", subcore_axis_name="s")


@functools.partial(
    pl.kernel,
    mesh=_mesh,
    out_type=jax.ShapeDtypeStruct((NC * NPAD, F), jnp.float32),
    scratch_types=[
        pltpu.VMEM((CHUNKS, K), jnp.int32),
        pltpu.VMEM((CHUNKS, K), jnp.int32),
        pltpu.VMEM((K, F), jnp.float32),
        pltpu.VMEM_SHARED((NPAD, F), jnp.float32),
        pltpu.SemaphoreType.DMA,
    ],
)
def _sc_scatter(feat_hbm, src_hbm, dst_hbm, zeros_hbm, out_hbm,
                src_all, dst_all, rows_v, acc_sh, sem):
    c = lax.axis_index("c")
    s = lax.axis_index("s")
    w = c * NS + s

    pltpu.sync_copy(src_hbm.at[w], src_all)
    pltpu.sync_copy(dst_hbm.at[w], dst_all)
    pltpu.sync_copy(zeros_hbm, acc_sh.at[pl.ds(s * ROWS_PER_TILE, ROWS_PER_TILE)])
    plsc.subcore_barrier()

    def body(_, j):
        pltpu.async_copy(feat_hbm.at[src_all.at[j]], rows_v, sem).wait()
        pltpu.sync_copy(rows_v, acc_sh.at[dst_all.at[j]], add=True)
        return j + jnp.int32(1)

    lax.fori_loop(0, CHUNKS, body, jnp.int32(0))
    plsc.subcore_barrier()

    row0 = s * ROWS_PER_TILE
    pltpu.sync_copy(acc_sh.at[pl.ds(row0, ROWS_PER_TILE)],
                    out_hbm.at[pl.ds(c * NPAD + row0, ROWS_PER_TILE)])


def _mm_body(p_ref, w_ref, o_ref):
    p0 = p_ref[pl.ds(0, N)]
    p1 = p_ref[pl.ds(NPAD, N)]
    o_ref[...] = jnp.dot(p0 + p1, w_ref[...],
                         preferred_element_type=jnp.float32)


def _combine_matmul(partial, W):
    return pl.pallas_call(
        _mm_body,
        out_shape=jax.ShapeDtypeStruct((N, F), jnp.float32),
    )(partial, W)


def kernel(features, edge_index, W, a):
    del a
    src = edge_index[0].astype(jnp.int32).reshape(NW, CHUNKS, K)
    dst = edge_index[1].astype(jnp.int32).reshape(NW, CHUNKS, K)
    zeros = jnp.zeros((ROWS_PER_TILE, F), jnp.float32)
    partial = _sc_scatter(features, src, dst, zeros)
    return _combine_matmul(partial, W)


# speedup vs baseline: 2.0049x; 1.1366x over previous
import functools

import jax
import jax.numpy as jnp
from jax import lax
from jax.experimental import pallas as pl
from jax.experimental.pallas import tpu as pltpu
from jax.experimental.pallas import tpu_sc as plsc

N = 10000
E = 320000
F = 128
NC = 2
NS = 16
NW = NC * NS
K = 125
EDGES_PER_WORKER = E // NW
CHUNKS = EDGES_PER_WORKER // K
NPAD = 10240
ROWS_PER_TILE = NPAD // NS

_mesh = plsc.VectorSubcoreMesh(core_axis_name="c", subcore_axis_name="s")


@functools.partial(
    pl.kernel,
    mesh=_mesh,
    out_type=jax.ShapeDtypeStruct((NC * NPAD, F), jnp.float32),
    scratch_types=[
        pltpu.VMEM((CHUNKS, K), jnp.int32),
        pltpu.VMEM((CHUNKS, K), jnp.int32),
        pltpu.VMEM((K, F), jnp.float32),
        pltpu.VMEM_SHARED((NPAD, F), jnp.float32),
        pltpu.SemaphoreType.DMA,
    ],
)
def _sc_scatter(feat_hbm, src_hbm, dst_hbm, zeros_hbm, out_hbm,
                src_all, dst_all, rows_v, acc_sh, sem):
    c = lax.axis_index("c")
    s = lax.axis_index("s")
    w = c * NS + s

    pltpu.sync_copy(src_hbm.at[w], src_all)
    pltpu.sync_copy(dst_hbm.at[w], dst_all)
    pltpu.sync_copy(zeros_hbm, acc_sh.at[pl.ds(s * ROWS_PER_TILE, ROWS_PER_TILE)])
    plsc.subcore_barrier()

    def body(_, j):
        pltpu.async_copy(feat_hbm.at[src_all.at[j]], rows_v, sem).wait()
        pltpu.sync_copy(rows_v, acc_sh.at[dst_all.at[j]], add=True)
        return j + jnp.int32(1)

    lax.fori_loop(0, CHUNKS, body, jnp.int32(0))
    plsc.subcore_barrier()

    row0 = s * ROWS_PER_TILE
    pltpu.sync_copy(acc_sh.at[pl.ds(row0, ROWS_PER_TILE)],
                    out_hbm.at[pl.ds(c * NPAD + row0, ROWS_PER_TILE)])


def _mm_body(p_ref, w_ref, o_ref):
    p0 = p_ref[pl.ds(0, N)]
    p1 = p_ref[pl.ds(NPAD, N)]
    o_ref[...] = jnp.dot(p0 + p1, w_ref[...],
                         preferred_element_type=jnp.float32)


def _combine_matmul(partial, W):
    return pl.pallas_call(
        _mm_body,
        out_shape=jax.ShapeDtypeStruct((N, F), jnp.float32),
    )(partial, W)


def kernel(features, edge_index, W, a):
    del a
    src = edge_index[0].astype(jnp.int32).reshape(NW, CHUNKS, K)
    dst = edge_index[1].astype(jnp.int32).reshape(NW, CHUNKS, K)
    zeros = jnp.zeros((ROWS_PER_TILE, F), jnp.float32)
    partial = _sc_scatter(features, src, dst, zeros)
    return _combine_matmul(partial, W)
